# Initial kernel scaffold; baseline (speedup 1.0000x reference)
import jax
import jax.numpy as jnp
from jax.experimental import pallas as pl


def kernel(inputs, rel_embeddings):
    raise NotImplementedError("write your pallas kernel here")


# speedup vs baseline: 8.9324x; 8.9324x over previous
import jax
import jax.numpy as jnp
from jax.experimental import pallas as pl

_MAX_POSITION = 2048
_DEPTH = 32
_ROWS_PER_BLOCK = 8


def _rpe_block(rev_ref, out_ref):
    i = pl.program_id(0)
    base = i * _ROWS_PER_BLOCK
    for r in range(_ROWS_PER_BLOCK):
        out_ref[r] = rev_ref[pl.ds(_MAX_POSITION - 1 - (base + r), _MAX_POSITION), :]


def kernel(inputs, rel_embeddings):
    length = inputs.shape[1]
    assert length == _MAX_POSITION and rel_embeddings.shape == (2 * _MAX_POSITION - 1, _DEPTH)
    rev = jnp.concatenate(
        [rel_embeddings[::-1], jnp.zeros((1, _DEPTH), rel_embeddings.dtype)], axis=0
    )
    grid = length // _ROWS_PER_BLOCK
    out = pl.pallas_call(
        _rpe_block,
        grid=(grid,),
        in_specs=[pl.BlockSpec((2 * _MAX_POSITION, _DEPTH), lambda i: (0, 0))],
        out_specs=pl.BlockSpec(
            (_ROWS_PER_BLOCK, _MAX_POSITION, _DEPTH), lambda i: (i, 0, 0)
        ),
        out_shape=jax.ShapeDtypeStruct((length, _MAX_POSITION, _DEPTH), jnp.float32),
    )(rev)
    return out


# speedup vs baseline: 13.6101x; 1.5237x over previous
import jax
import jax.numpy as jnp
from jax.experimental import pallas as pl
from jax.experimental.pallas import tpu as pltpu

_L = 2048
_DEPTH = 32
_ROWS_PER_STEP = 8


def _rpe_dma(p_ref, out_ref, sem):
    i = pl.program_id(0)
    base = i * _ROWS_PER_STEP
    for r in range(_ROWS_PER_STEP):
        row = base + r
        s = _L - 1 - row
        ph = (3 - r) % 4
        q = jax.lax.div(s, 4)
        pltpu.make_async_copy(
            p_ref.at[ph, pl.ds(q, _L // 4), :],
            out_ref.at[row],
            sem,
        ).start()
    for _ in range(_ROWS_PER_STEP):
        pltpu.make_async_copy(
            p_ref.at[0, pl.ds(0, _L // 4), :],
            out_ref.at[0],
            sem,
        ).wait()


def kernel(inputs, rel_embeddings):
    length = inputs.shape[1]
    assert length == _L and rel_embeddings.shape == (2 * _L - 1, _DEPTH)
    rev_flat = rel_embeddings[::-1].reshape(-1)
    rev_flat = jnp.concatenate(
        [rev_flat, jnp.zeros((_L * 2 * _DEPTH + 96 - rev_flat.shape[0],), rev_flat.dtype)]
    )
    phases = jnp.stack(
        [rev_flat[32 * p : 32 * p + _L * 2 * _DEPTH].reshape(_L // 2, 128) for p in range(4)]
    )
    out = pl.pallas_call(
        _rpe_dma,
        grid=(length // _ROWS_PER_STEP,),
        in_specs=[pl.BlockSpec((4, _L // 2, 128), lambda i: (0, 0, 0))],
        out_specs=pl.BlockSpec(memory_space=pltpu.MemorySpace.HBM),
        out_shape=jax.ShapeDtypeStruct((length, _L // 4, 128), jnp.float32),
        scratch_shapes=[pltpu.SemaphoreType.DMA],
        compiler_params=pltpu.CompilerParams(
            dimension_semantics=("arbitrary",),
        ),
    )(phases)
    return out.reshape(length, length, _DEPTH)


# speedup vs baseline: 15.4165x; 1.1327x over previous
import jax
import jax.numpy as jnp
from jax.experimental import pallas as pl
from jax.experimental.pallas import tpu as pltpu

_L = 2048
_DEPTH = 32
_ROWS_PER_STEP = 32
_ROW_VREGS = _L * _DEPTH // 128


def _rpe_copy(t_ref, out_ref):
    g = pl.program_id(0)
    q = 504 - 8 * g
    for r in range(_ROWS_PER_STEP):
        out_ref[r] = t_ref[31 - r, pl.ds(q, _ROW_VREGS), :]


def kernel(inputs, rel_embeddings):
    length = inputs.shape[1]
    assert length == _L and rel_embeddings.shape == (2 * _L - 1, _DEPTH)
    rev_flat = rel_embeddings[::-1].reshape(-1)
    pad = 32 * 31 + _L * 2 * _DEPTH - rev_flat.shape[0]
    rev_flat = jnp.concatenate([rev_flat, jnp.zeros((pad,), rev_flat.dtype)])
    tables = jnp.stack(
        [rev_flat[32 * m : 32 * m + _L * 2 * _DEPTH].reshape(_L // 2, 128) for m in range(32)]
    )
    out = pl.pallas_call(
        _rpe_copy,
        grid=(length // _ROWS_PER_STEP,),
        in_specs=[pl.BlockSpec((32, _L // 2, 128), lambda g: (0, 0, 0))],
        out_specs=pl.BlockSpec((_ROWS_PER_STEP, _ROW_VREGS, 128), lambda g: (g, 0, 0)),
        out_shape=jax.ShapeDtypeStruct((length, _ROW_VREGS, 128), jnp.float32),
        compiler_params=pltpu.CompilerParams(
            dimension_semantics=("arbitrary",),
        ),
    )(tables)
    return out.reshape(length, length, _DEPTH)


# speedup vs baseline: 94.0283x; 6.0992x over previous
import jax
import jax.numpy as jnp
from jax.experimental import pallas as pl
from jax.experimental.pallas import tpu as pltpu

_L = 2048
_DEPTH = 32
_PHASES = 128
_PH_PER_STEP = 2
_ROWS_PER_PHASE = _L // _PHASES


def _rpe_phase(rev_ref, out_ref):
    g = pl.program_id(0)
    for u in range(_PH_PER_STEP):
        c = g * _PH_PER_STEP + u
        m = _PHASES - 1 - c
        for dg in range(_DEPTH // 8):
            slab = pltpu.roll(rev_ref[pl.ds(8 * dg, 8), :], -m, axis=1)
            for t in range(_ROWS_PER_PHASE):
                start = _PHASES * (_ROWS_PER_PHASE - 1 - t)
                out_ref[t, u, pl.ds(8 * dg, 8), :] = slab[:, start : start + _L]


def kernel(inputs, rel_embeddings):
    length = inputs.shape[1]
    assert length == _L and rel_embeddings.shape == (2 * _L - 1, _DEPTH)
    revT = jnp.zeros((_DEPTH, 2 * _L), jnp.float32).at[:, : 2 * _L - 1].set(
        rel_embeddings[::-1].T
    )
    out4 = pl.pallas_call(
        _rpe_phase,
        grid=(_PHASES // _PH_PER_STEP,),
        in_specs=[pl.BlockSpec((_DEPTH, 2 * _L), lambda g: (0, 0))],
        out_specs=pl.BlockSpec(
            (_ROWS_PER_PHASE, _PH_PER_STEP, _DEPTH, _L), lambda g: (0, g, 0, 0)
        ),
        out_shape=jax.ShapeDtypeStruct((_ROWS_PER_PHASE, _PHASES, _DEPTH, _L), jnp.float32),
        compiler_params=pltpu.CompilerParams(
            dimension_semantics=("arbitrary",),
        ),
    )(revT)
    return out4.reshape(length, _DEPTH, length).transpose(0, 2, 1)


# speedup vs baseline: 95.8810x; 1.0197x over previous
import jax
import jax.numpy as jnp
from jax.experimental import pallas as pl
from jax.experimental.pallas import tpu as pltpu

_L = 2048
_DEPTH = 32
_PHASES = 128
_ROWS_PER_PHASE = _L // _PHASES


def _rpe_phase(rev_ref, out_ref):
    c = pl.program_id(0)
    m = _PHASES - 1 - c
    for dg in range(_DEPTH // 8):
        slab = pltpu.roll(rev_ref[pl.ds(8 * dg, 8), :], -m, axis=1)
        for t in range(_ROWS_PER_PHASE):
            start = _PHASES * (_ROWS_PER_PHASE - 1 - t)
            out_ref[t, 0, pl.ds(8 * dg, 8), :] = slab[:, start : start + _L]


def kernel(inputs, rel_embeddings):
    length = inputs.shape[1]
    assert length == _L and rel_embeddings.shape == (2 * _L - 1, _DEPTH)
    revT = jnp.zeros((_DEPTH, 2 * _L), jnp.float32).at[:, : 2 * _L - 1].set(
        rel_embeddings[::-1].T
    )
    out4 = pl.pallas_call(
        _rpe_phase,
        grid=(_PHASES,),
        in_specs=[pl.BlockSpec((_DEPTH, 2 * _L), lambda c: (0, 0))],
        out_specs=pl.BlockSpec(
            (_ROWS_PER_PHASE, 1, _DEPTH, _L), lambda c: (0, c, 0, 0)
        ),
        out_shape=jax.ShapeDtypeStruct((_ROWS_PER_PHASE, _PHASES, _DEPTH, _L), jnp.float32),
        compiler_params=pltpu.CompilerParams(
            dimension_semantics=("arbitrary",),
        ),
    )(revT)
    return out4.reshape(length, _DEPTH, length).transpose(0, 2, 1)
